# baseline (speedup 1.0000x reference)
import functools
import math

import jax
import jax.numpy as jnp
from jax import lax
from jax.experimental import pallas as pl
from jax.experimental.pallas import tpu as pltpu
from jax.experimental.pallas import tpu_sc as plsc

_NC = 2
_NS = 16
_NW = _NC * _NS
_L = 16

_B = 16384
_D = 64
_BPW = _B // _NW
_GCH = 128
_NG = _BPW // _GCH
_NCH = _D // _L

_BN_SCALE = 1.0 / math.sqrt(1.0 + 1e-5)

_mesh = plsc.VectorSubcoreMesh(core_axis_name="c", subcore_axis_name="s")

_GATHER_DNUMS = lax.GatherDimensionNumbers(
    offset_dims=(), collapsed_slice_dims=(0,), start_index_map=(0,))


def _lane_perm(v, idx):
    return lax.gather(v, idx[:, None], _GATHER_DNUMS, slice_sizes=(1,),
                      mode=lax.GatherScatterMode.PROMISE_IN_BOUNDS)


@functools.partial(
    pl.kernel,
    mesh=_mesh,
    compiler_params=pltpu.CompilerParams(use_tc_tiling_on_sc=False),
    out_type=jax.ShapeDtypeStruct((_B,), jnp.float32),
    scratch_types=[
        pltpu.VMEM((_NG, _GCH), jnp.int32),
        pltpu.VMEM((_NG, _GCH), jnp.int32),
        pltpu.VMEM((_BPW, _D), jnp.float32),
        pltpu.VMEM((_BPW, _D), jnp.float32),
        pltpu.VMEM((_BPW,), jnp.float32),
        pltpu.VMEM((_BPW,), jnp.float32),
        pltpu.VMEM((_BPW,), jnp.float32),
        pltpu.VMEM((4 * _D + _L,), jnp.float32),
        pltpu.SemaphoreType.DMA,
        pltpu.SemaphoreType.DMA,
    ],
)
def _mf_kernel(uids, iids, uemb, iemb, ubias, ibias, consts, out,
               idx_u, idx_v, rows_u, rows_v, gbu, gbv, out_v,
               consts_v, sem_small, sem_rows):
    wid = lax.axis_index("s") * _NC + lax.axis_index("c")
    base = wid * _BPW

    small = [pltpu.async_copy(consts, consts_v, sem_small)]
    for j in range(_NG):
        small.append(pltpu.async_copy(
            uids.at[pl.ds(base + j * _GCH, _GCH)], idx_u.at[j], sem_small))
        small.append(pltpu.async_copy(
            iids.at[pl.ds(base + j * _GCH, _GCH)], idx_v.at[j], sem_small))
    for cp in small:
        cp.wait()

    gathers = []
    for j in range(_NG):
        sl = pl.ds(j * _GCH, _GCH)
        gathers.append(pltpu.async_copy(uemb.at[idx_u.at[j]], rows_u.at[sl], sem_rows))
        gathers.append(pltpu.async_copy(iemb.at[idx_v.at[j]], rows_v.at[sl], sem_rows))
        gathers.append(pltpu.async_copy(ubias.at[idx_u.at[j]], gbu.at[sl], sem_small))
        gathers.append(pltpu.async_copy(ibias.at[idx_v.at[j]], gbv.at[sl], sem_small))
    for cp in gathers:
        cp.wait()

    sus = [consts_v[pl.ds(c * _L, _L)] for c in range(_NCH)]
    bus = [consts_v[pl.ds(_D + c * _L, _L)] for c in range(_NCH)]
    svs = [consts_v[pl.ds(2 * _D + c * _L, _L)] for c in range(_NCH)]
    bvs = [consts_v[pl.ds(3 * _D + c * _L, _L)] for c in range(_NCH)]
    gb = consts_v[pl.ds(4 * _D, _L)]

    iota = lax.iota(jnp.int32, _L)
    perms = [jnp.bitwise_xor(iota, sh) for sh in (8, 4, 2, 1)]
    zero = gb * 0.0
    def body_g(g, carry):
        r = zero
        for j in range(_L):
            i = g * _L + j
            acc = None
            for c in range(_NCH):
                u = rows_u[i, pl.ds(c * _L, _L)]
                v = rows_v[i, pl.ds(c * _L, _L)]
                ub = u * sus[c] + bus[c]
                vb = v * svs[c] + bvs[c]
                p = ub * vb
                acc = p if acc is None else acc + p
            for pm in perms:
                acc = acc + _lane_perm(acc, pm)
            r = jnp.where(iota == j, acc, r)
        sl = pl.ds(g * _L, _L)
        out_v[sl] = r + gbu[sl] + gbv[sl] + gb
        return carry
    lax.fori_loop(0, _BPW // _L, body_g, 0)

    pltpu.sync_copy(out_v, out.at[pl.ds(base, _BPW)])


def kernel(user_ids, item_ids, user_emb, item_emb, user_bias_tab,
           item_bias_tab, global_bias, user_bn_gamma, user_bn_beta,
           item_bn_gamma, item_bn_beta):
    su = user_bn_gamma * _BN_SCALE
    sv = item_bn_gamma * _BN_SCALE
    consts = jnp.concatenate([
        su, user_bn_beta, sv, item_bn_beta,
        jnp.broadcast_to(global_bias, (_L,)),
    ]).astype(jnp.float32)
    return _mf_kernel(user_ids.astype(jnp.int32), item_ids.astype(jnp.int32),
                      user_emb, item_emb,
                      user_bias_tab.reshape(-1), item_bias_tab.reshape(-1),
                      consts)


# speedup vs baseline: 3.4089x; 3.4089x over previous
import functools
import math

import jax
import jax.numpy as jnp
from jax import lax
from jax.experimental import pallas as pl
from jax.experimental.pallas import tpu as pltpu
from jax.experimental.pallas import tpu_sc as plsc

_NC = 2
_NS = 16
_NW = _NC * _NS
_L = 16

_B = 16384
_D = 64
_BPW = _B // _NW
_CH = 4
_NCHK = _BPW // _CH
_NSLOT = 14
_RW = 2 * _D

_BN_SCALE = 1.0 / math.sqrt(1.0 + 1e-5)

_mesh = plsc.VectorSubcoreMesh(core_axis_name="c", subcore_axis_name="s")


def _expand8(a):
    a4 = a.reshape(-1, _CH)
    pad = jnp.zeros_like(a4)
    return jnp.concatenate([a4, pad], axis=1).reshape(-1)


@functools.partial(
    pl.kernel,
    mesh=_mesh,
    compiler_params=pltpu.CompilerParams(
        use_tc_tiling_on_sc=True, needs_layout_passes=False),
    out_type=(jax.ShapeDtypeStruct((_B, _RW), jnp.float32),
              jax.ShapeDtypeStruct((_B, _RW), jnp.float32)),
    scratch_types=[
        pltpu.VMEM((_NSLOT, _D, 128), jnp.float32),
        pltpu.VMEM((_NSLOT, 1, 128), jnp.float32),
        pltpu.VMEM((2, 2 * _CH, _RW), jnp.float32),
        pltpu.VMEM((2 * _BPW + _L,), jnp.int32),
        pltpu.VMEM((2 * _BPW + _L,), jnp.int32),
        pltpu.VMEM((2 * _BPW + _L,), jnp.int32),
        pltpu.SemaphoreType.DMA,
        pltpu.SemaphoreType.DMA,
        pltpu.SemaphoreType.DMA,
    ],
)
def _gather_kernel(uembT, ubiasT, iembT, ibiasT, suids, uslot, unew,
                   siids, islot, inew, gu, gi,
                   bufs, bbufs, stage, idv, slv, nwv, sem_b, sem_s, sem_o):
    wid = lax.axis_index("s") * _NC + lax.axis_index("c")
    iota = lax.iota(jnp.int32, _L)

    def phase(embT, biasT, sids_h, slot_h, new_h, gout):
        hb = wid * (2 * _BPW)
        small = [
            pltpu.async_copy(sids_h.at[pl.ds(hb, 2 * _BPW)],
                             idv.at[pl.ds(0, 2 * _BPW)], sem_s),
            pltpu.async_copy(slot_h.at[pl.ds(hb, 2 * _BPW)],
                             slv.at[pl.ds(0, 2 * _BPW)], sem_s),
            pltpu.async_copy(new_h.at[pl.ds(hb, 2 * _BPW)],
                             nwv.at[pl.ds(0, 2 * _BPW)], sem_s),
        ]
        for cp in small:
            cp.wait()

        def body(m, n_prev):
            ids16 = idv[pl.ds(m * 2 * _CH, _L)]
            new16 = nwv[pl.ds(m * 2 * _CH, _L)]
            slt16 = slv[pl.ds(m * 2 * _CH, _L)]
            issue_ok = m < _NCHK
            n_m = jnp.int32(0)
            for j in range(_CH):
                blk = lax.shift_right_logical(ids16[j], 7)
                off = pl.multiple_of(blk * 128, 128)
                slot = slt16[j]
                fresh = new16[j]

                @pl.when(jnp.logical_and(issue_ok, fresh == 1))
                def _():
                    pltpu.async_copy(embT.at[:, pl.ds(off, 128)],
                                     bufs.at[slot], sem_b)
                    pltpu.async_copy(biasT.at[:, pl.ds(off, 128)],
                                     bbufs.at[slot], sem_b)
                n_m = n_m + fresh
            n_m = jnp.where(issue_ok, n_m, 0)

            def drain(_, c):
                pltpu.make_async_copy(embT.at[:, pl.ds(0, 128)],
                                      bufs.at[0], sem_b).wait()
                pltpu.make_async_copy(biasT.at[:, pl.ds(0, 128)],
                                      bbufs.at[0], sem_b).wait()
                return c
            lax.fori_loop(0, n_prev, drain, 0)

            @pl.when(m >= 1)
            def _():
                e = m - 1
                pair = lax.shift_right_logical(e, 1)
                pslot = lax.bitwise_and(pair, 1)
                jbase = lax.bitwise_and(e, 1) * _CH

                @pl.when(jnp.logical_and(lax.bitwise_and(e, 1) == 0,
                                         pair >= 2))
                def _():
                    pltpu.make_async_copy(
                        embT.at[pl.ds(0, 8), pl.ds(0, 128)],
                        stage.at[0], sem_o).wait()

                eids = idv[pl.ds(e * 2 * _CH, _L)]
                eslt = slv[pl.ds(e * 2 * _CH, _L)]
                for j in range(_CH):
                    slot = jnp.broadcast_to(eslt[j], (_L,))
                    col = jnp.broadcast_to(
                        lax.bitwise_and(eids[j], 127), (_L,))
                    for c in range(_D // _L):
                        g = plsc.load_gather(
                            bufs, [slot, iota + c * _L, col])
                        stage[pslot, jbase + j, pl.ds(c * _L, _L)] = g
                    bg = plsc.load_gather(
                        bbufs, [slot, jnp.broadcast_to(0, (_L,)), col])
                    stage[pslot, jbase + j, pl.ds(_D, _L)] = bg

                @pl.when(lax.bitwise_and(e, 1) == 1)
                def _():
                    pltpu.async_copy(
                        stage.at[pslot],
                        gout.at[pl.ds(wid * _BPW + pair * 2 * _CH, 2 * _CH)],
                        sem_o)
            return n_m
        lax.fori_loop(0, _NCHK + 1, body, jnp.int32(0))
        for _ in range(2):
            pltpu.make_async_copy(embT.at[pl.ds(0, 8), pl.ds(0, 128)],
                                  stage.at[0], sem_o).wait()

    phase(uembT, ubiasT, suids, uslot, unew, gu)
    phase(iembT, ibiasT, siids, islot, inew, gi)


_GCH = 128
_NG2 = _BPW // _GCH

_GATHER_DNUMS = lax.GatherDimensionNumbers(
    offset_dims=(), collapsed_slice_dims=(0,), start_index_map=(0,))


def _lane_perm(v, idx):
    return lax.gather(v, idx[:, None], _GATHER_DNUMS, slice_sizes=(1,),
                      mode=lax.GatherScatterMode.PROMISE_IN_BOUNDS)


@functools.partial(
    pl.kernel,
    mesh=_mesh,
    out_type=jax.ShapeDtypeStruct((_B,), jnp.float32),
    scratch_types=[
        pltpu.VMEM((_NG2, _GCH), jnp.int32),
        pltpu.VMEM((_NG2, _GCH), jnp.int32),
        pltpu.VMEM((_BPW // 2, _RW), jnp.float32),
        pltpu.VMEM((_BPW // 2, _RW), jnp.float32),
        pltpu.VMEM((_BPW,), jnp.float32),
        pltpu.VMEM((3 * _D + _L,), jnp.float32),
        pltpu.SemaphoreType.DMA,
        pltpu.SemaphoreType.DMA,
    ],
)
def _dot_kernel(gu, gi, uipu, iipu, consts, out,
                idx_u, idx_v, rows_u, rows_v, out_v, consts_v,
                sem_small, sem_rows):
    wid = lax.axis_index("s") * _NC + lax.axis_index("c")
    base = wid * _BPW

    small = [pltpu.async_copy(consts, consts_v, sem_small)]
    for j in range(_NG2):
        small.append(pltpu.async_copy(
            uipu.at[pl.ds(base + j * _GCH, _GCH)], idx_u.at[j], sem_small))
        small.append(pltpu.async_copy(
            iipu.at[pl.ds(base + j * _GCH, _GCH)], idx_v.at[j], sem_small))
    for cp in small:
        cp.wait()

    wvecs = [consts_v[pl.ds(c * _L, _L)] for c in range(_D // _L)]
    pvecs = [consts_v[pl.ds(_D + c * _L, _L)] for c in range(_D // _L)]
    qvecs = [consts_v[pl.ds(2 * _D + c * _L, _L)] for c in range(_D // _L)]
    cvec = consts_v[pl.ds(3 * _D, _L)]

    iota = lax.iota(jnp.int32, _L)
    perms = [jnp.bitwise_xor(iota, sh) for sh in (8, 4, 2, 1)]
    zero = cvec * 0.0

    for half in range(2):
        gathers = []
        for j in range(_NG2 // 2):
            sl = pl.ds(j * _GCH, _GCH)
            jj = half * (_NG2 // 2) + j
            gathers.append(pltpu.async_copy(
                gu.at[idx_u.at[jj]], rows_u.at[sl], sem_rows))
            gathers.append(pltpu.async_copy(
                gi.at[idx_v.at[jj]], rows_v.at[sl], sem_rows))
        for cp in gathers:
            cp.wait()

        def body_g(g, carry):
            r = zero
            for j in range(_L):
                i = g * _L + j
                acc = None
                for c in range(_D // _L):
                    u = rows_u[i, pl.ds(c * _L, _L)]
                    v = rows_v[i, pl.ds(c * _L, _L)]
                    t = u * v * wvecs[c] + u * pvecs[c] + v * qvecs[c]
                    acc = t if acc is None else acc + t
                for pm in perms:
                    acc = acc + _lane_perm(acc, pm)
                ub = rows_u[i, pl.ds(_D, _L)]
                vb = rows_v[i, pl.ds(_D, _L)]
                r = jnp.where(iota == j, acc + ub + vb + cvec, r)
            out_v[pl.ds(half * (_BPW // 2) + g * _L, _L)] = r
            return carry
        lax.fori_loop(0, _BPW // 2 // _L, body_g, 0)

    pltpu.sync_copy(out_v, out.at[pl.ds(base, _BPW)])


def kernel(user_ids, item_ids, user_emb, item_emb, user_bias_tab,
           item_bias_tab, global_bias, user_bn_gamma, user_bn_beta,
           item_bn_gamma, item_bn_beta):
    uids = user_ids.astype(jnp.int32)
    iids = item_ids.astype(jnp.int32)

    def route(ids):
        p = jnp.argsort(ids)
        sids = ids[p]
        blocks = lax.shift_right_logical(sids, 7)
        is_new = jnp.concatenate([
            jnp.ones((1,), jnp.int32),
            (blocks[1:] != blocks[:-1]).astype(jnp.int32)])
        is_new = jnp.maximum(
            is_new, (jnp.arange(_B, dtype=jnp.int32) % _BPW == 0)
            .astype(jnp.int32))
        slot = (jnp.cumsum(is_new) - 1).astype(jnp.int32) % _NSLOT
        inv = jnp.argsort(p).astype(jnp.int32)
        return (_expand8(sids), _expand8(slot), _expand8(is_new), inv)

    suids, uslot, unew, uinv = route(uids)
    siids, islot, inew, iinv = route(iids)

    su = user_bn_gamma * _BN_SCALE
    sv = item_bn_gamma * _BN_SCALE
    w = su * sv
    p = su * item_bn_beta
    q = sv * user_bn_beta
    const_term = jnp.dot(user_bn_beta, item_bn_beta) + global_bias[0]
    consts = jnp.concatenate(
        [w, p, q, jnp.broadcast_to(const_term, (_L,))]).astype(jnp.float32)

    gu, gi = _gather_kernel(user_emb.T, user_bias_tab.T,
                            item_emb.T, item_bias_tab.T,
                            suids, uslot, unew, siids, islot, inew)
    return _dot_kernel(gu, gi, uinv, iinv, consts)


# speedup vs baseline: 4.0172x; 1.1784x over previous
import functools
import math

import jax
import jax.numpy as jnp
from jax import lax
from jax.experimental import pallas as pl
from jax.experimental.pallas import tpu as pltpu
from jax.experimental.pallas import tpu_sc as plsc

_NC = 2
_NS = 16
_NW = _NC * _NS
_L = 16

_B = 16384
_D = 64
_BPW = _B // _NW
_CH = 4
_NCHK = _BPW // _CH
_NSLOT = 14
_RW = 2 * _D

_BN_SCALE = 1.0 / math.sqrt(1.0 + 1e-5)

_mesh = plsc.VectorSubcoreMesh(core_axis_name="c", subcore_axis_name="s")


def _expand8(a):
    a4 = a.reshape(-1, _CH)
    pad = jnp.zeros_like(a4)
    return jnp.concatenate([a4, pad], axis=1).reshape(-1)


@functools.partial(
    pl.kernel,
    mesh=_mesh,
    compiler_params=pltpu.CompilerParams(
        use_tc_tiling_on_sc=True, needs_layout_passes=False),
    out_type=(jax.ShapeDtypeStruct((_B, _RW), jnp.float32),
              jax.ShapeDtypeStruct((_B, _RW), jnp.float32)),
    scratch_types=[
        pltpu.VMEM((_NSLOT, _D, 128), jnp.float32),
        pltpu.VMEM((_NSLOT, 1, 128), jnp.float32),
        pltpu.VMEM((2, 2 * _CH, _RW), jnp.float32),
        pltpu.VMEM((2 * _BPW + 2 * _L,), jnp.int32),
        pltpu.VMEM((2 * _BPW + 2 * _L,), jnp.int32),
        pltpu.VMEM((2 * _BPW + 2 * _L,), jnp.int32),
        pltpu.SemaphoreType.DMA,
        pltpu.SemaphoreType.DMA,
        pltpu.SemaphoreType.DMA,
    ],
)
def _gather_kernel(uembT, ubiasT, iembT, ibiasT, suids, uslot, unew,
                   siids, islot, inew, gu, gi,
                   bufs, bbufs, stage, idv, slv, nwv, sem_b, sem_s, sem_o):
    wid = lax.axis_index("s") * _NC + lax.axis_index("c")
    iota = lax.iota(jnp.int32, _L)

    def phase(embT, biasT, sids_h, slot_h, new_h, gout):
        hb = wid * (2 * _BPW)
        small = [
            pltpu.async_copy(sids_h.at[pl.ds(hb, 2 * _BPW)],
                             idv.at[pl.ds(0, 2 * _BPW)], sem_s),
            pltpu.async_copy(slot_h.at[pl.ds(hb, 2 * _BPW)],
                             slv.at[pl.ds(0, 2 * _BPW)], sem_s),
            pltpu.async_copy(new_h.at[pl.ds(hb, 2 * _BPW)],
                             nwv.at[pl.ds(0, 2 * _BPW)], sem_s),
        ]
        for cp in small:
            cp.wait()

        def body(m, n_prev):
            ids16 = idv[pl.ds(m * 2 * _CH, _L)]
            new16 = nwv[pl.ds(m * 2 * _CH, _L)]
            slt16 = slv[pl.ds(m * 2 * _CH, _L)]
            issue_ok = m < _NCHK
            n_m = jnp.int32(0)
            for j in range(_CH):
                blk = lax.shift_right_logical(ids16[j], 7)
                off = pl.multiple_of(blk * 128, 128)
                slot = slt16[j]
                fresh = new16[j]

                @pl.when(jnp.logical_and(issue_ok, fresh == 1))
                def _():
                    pltpu.async_copy(embT.at[:, pl.ds(off, 128)],
                                     bufs.at[slot], sem_b)
                    pltpu.async_copy(biasT.at[:, pl.ds(off, 128)],
                                     bbufs.at[slot], sem_b)
                n_m = n_m + fresh
            n_m = jnp.where(issue_ok, n_m, 0)

            n_prev1, n_prev2 = n_prev

            def drain(_, c):
                pltpu.make_async_copy(embT.at[:, pl.ds(0, 128)],
                                      bufs.at[0], sem_b).wait()
                pltpu.make_async_copy(biasT.at[:, pl.ds(0, 128)],
                                      bbufs.at[0], sem_b).wait()
                return c
            lax.fori_loop(0, n_prev2, drain, 0)

            @pl.when(m >= 2)
            def _():
                e = m - 2
                pair = lax.shift_right_logical(e, 1)
                pslot = lax.bitwise_and(pair, 1)
                jbase = lax.bitwise_and(e, 1) * _CH

                @pl.when(jnp.logical_and(lax.bitwise_and(e, 1) == 0,
                                         pair >= 2))
                def _():
                    pltpu.make_async_copy(
                        embT.at[pl.ds(0, 8), pl.ds(0, 128)],
                        stage.at[0], sem_o).wait()

                eids = idv[pl.ds(e * 2 * _CH, _L)]
                eslt = slv[pl.ds(e * 2 * _CH, _L)]
                for j in range(_CH):
                    slot = jnp.broadcast_to(eslt[j], (_L,))
                    col = jnp.broadcast_to(
                        lax.bitwise_and(eids[j], 127), (_L,))
                    for c in range(_D // _L):
                        g = plsc.load_gather(
                            bufs, [slot, iota + c * _L, col])
                        stage[pslot, jbase + j, pl.ds(c * _L, _L)] = g
                    bg = plsc.load_gather(
                        bbufs, [slot, jnp.broadcast_to(0, (_L,)), col])
                    stage[pslot, jbase + j, pl.ds(_D, _L)] = bg

                @pl.when(lax.bitwise_and(e, 1) == 1)
                def _():
                    pltpu.async_copy(
                        stage.at[pslot],
                        gout.at[pl.ds(wid * _BPW + pair * 2 * _CH, 2 * _CH)],
                        sem_o)
            return (n_m, n_prev1)
        lax.fori_loop(0, _NCHK + 2, body, (jnp.int32(0), jnp.int32(0)))
        for _ in range(2):
            pltpu.make_async_copy(embT.at[pl.ds(0, 8), pl.ds(0, 128)],
                                  stage.at[0], sem_o).wait()

    phase(uembT, ubiasT, suids, uslot, unew, gu)
    phase(iembT, ibiasT, siids, islot, inew, gi)


_GCH = 128
_NG2 = _BPW // _GCH

_GATHER_DNUMS = lax.GatherDimensionNumbers(
    offset_dims=(), collapsed_slice_dims=(0,), start_index_map=(0,))


def _lane_perm(v, idx):
    return lax.gather(v, idx[:, None], _GATHER_DNUMS, slice_sizes=(1,),
                      mode=lax.GatherScatterMode.PROMISE_IN_BOUNDS)


@functools.partial(
    pl.kernel,
    mesh=_mesh,
    out_type=jax.ShapeDtypeStruct((_B,), jnp.float32),
    scratch_types=[
        pltpu.VMEM((_NG2, _GCH), jnp.int32),
        pltpu.VMEM((_NG2, _GCH), jnp.int32),
        pltpu.VMEM((_BPW // 2, _RW), jnp.float32),
        pltpu.VMEM((_BPW // 2, _RW), jnp.float32),
        pltpu.VMEM((_BPW,), jnp.float32),
        pltpu.VMEM((3 * _D + _L,), jnp.float32),
        pltpu.SemaphoreType.DMA,
        pltpu.SemaphoreType.DMA,
    ],
)
def _dot_kernel(gu, gi, uipu, iipu, consts, out,
                idx_u, idx_v, rows_u, rows_v, out_v, consts_v,
                sem_small, sem_rows):
    wid = lax.axis_index("s") * _NC + lax.axis_index("c")
    base = wid * _BPW

    small = [pltpu.async_copy(consts, consts_v, sem_small)]
    for j in range(_NG2):
        small.append(pltpu.async_copy(
            uipu.at[pl.ds(base + j * _GCH, _GCH)], idx_u.at[j], sem_small))
        small.append(pltpu.async_copy(
            iipu.at[pl.ds(base + j * _GCH, _GCH)], idx_v.at[j], sem_small))
    for cp in small:
        cp.wait()

    wvecs = [consts_v[pl.ds(c * _L, _L)] for c in range(_D // _L)]
    pvecs = [consts_v[pl.ds(_D + c * _L, _L)] for c in range(_D // _L)]
    qvecs = [consts_v[pl.ds(2 * _D + c * _L, _L)] for c in range(_D // _L)]
    cvec = consts_v[pl.ds(3 * _D, _L)]

    iota = lax.iota(jnp.int32, _L)
    perms = [jnp.bitwise_xor(iota, sh) for sh in (8, 4, 2, 1)]
    zero = cvec * 0.0

    for half in range(2):
        gathers = []
        for j in range(_NG2 // 2):
            sl = pl.ds(j * _GCH, _GCH)
            jj = half * (_NG2 // 2) + j
            gathers.append(pltpu.async_copy(
                gu.at[idx_u.at[jj]], rows_u.at[sl], sem_rows))
            gathers.append(pltpu.async_copy(
                gi.at[idx_v.at[jj]], rows_v.at[sl], sem_rows))
        for cp in gathers:
            cp.wait()

        def body_g(g, carry):
            r = zero
            for j in range(_L):
                i = g * _L + j
                acc = None
                for c in range(_D // _L):
                    u = rows_u[i, pl.ds(c * _L, _L)]
                    v = rows_v[i, pl.ds(c * _L, _L)]
                    t = u * v * wvecs[c] + u * pvecs[c] + v * qvecs[c]
                    acc = t if acc is None else acc + t
                for pm in perms:
                    acc = acc + _lane_perm(acc, pm)
                ub = rows_u[i, pl.ds(_D, _L)]
                vb = rows_v[i, pl.ds(_D, _L)]
                r = jnp.where(iota == j, acc + ub + vb + cvec, r)
            out_v[pl.ds(half * (_BPW // 2) + g * _L, _L)] = r
            return carry
        lax.fori_loop(0, _BPW // 2 // _L, body_g, 0)

    pltpu.sync_copy(out_v, out.at[pl.ds(base, _BPW)])


def kernel(user_ids, item_ids, user_emb, item_emb, user_bias_tab,
           item_bias_tab, global_bias, user_bn_gamma, user_bn_beta,
           item_bn_gamma, item_bn_beta):
    uids = user_ids.astype(jnp.int32)
    iids = item_ids.astype(jnp.int32)

    def route(ids):
        p = jnp.argsort(ids)
        sids = ids[p]
        blocks = lax.shift_right_logical(sids, 7)
        is_new = jnp.concatenate([
            jnp.ones((1,), jnp.int32),
            (blocks[1:] != blocks[:-1]).astype(jnp.int32)])
        is_new = jnp.maximum(
            is_new, (jnp.arange(_B, dtype=jnp.int32) % _BPW == 0)
            .astype(jnp.int32))
        slot = (jnp.cumsum(is_new) - 1).astype(jnp.int32) % _NSLOT
        inv = jnp.argsort(p).astype(jnp.int32)
        return (_expand8(sids), _expand8(slot), _expand8(is_new), inv)

    suids, uslot, unew, uinv = route(uids)
    siids, islot, inew, iinv = route(iids)

    su = user_bn_gamma * _BN_SCALE
    sv = item_bn_gamma * _BN_SCALE
    w = su * sv
    p = su * item_bn_beta
    q = sv * user_bn_beta
    const_term = jnp.dot(user_bn_beta, item_bn_beta) + global_bias[0]
    consts = jnp.concatenate(
        [w, p, q, jnp.broadcast_to(const_term, (_L,))]).astype(jnp.float32)

    gu, gi = _gather_kernel(user_emb.T, user_bias_tab.T,
                            item_emb.T, item_bias_tab.T,
                            suids, uslot, unew, siids, islot, inew)
    return _dot_kernel(gu, gi, uinv, iinv, consts)
